# baseline (speedup 1.0000x reference)
import functools

import jax
import jax.numpy as jnp
from jax import lax
from jax.experimental import pallas as pl
from jax.experimental.pallas import tpu as pltpu
from jax.experimental.pallas import tpu_sc as plsc

MAX_LEN = 8192
EMB = 64

NC = 2
NS = 16
NW = NC * NS

CHUNK = 128
NBUF = 8


def _fuse_body(w_ref, p_ref, o_ref):
    o_ref[...] = w_ref[...] + p_ref[...]


def _fuse_tables(word_head, pos_table):
    return pl.pallas_call(
        _fuse_body,
        out_shape=jax.ShapeDtypeStruct((MAX_LEN, EMB), jnp.float32),
    )(word_head, pos_table)


def _gather_kernel(n_tokens):
    assert n_tokens % (NW * CHUNK) == 0
    per_w = n_tokens // NW
    n_chunks = per_w // CHUNK
    S = 5
    assert n_chunks % (2 * S) == 0
    n_groups = n_chunks // S
    pairs = n_groups // 2
    GROUP_ROWS = S * CHUNK

    mesh = plsc.VectorSubcoreMesh(core_axis_name="c", subcore_axis_name="s")

    @functools.partial(
        pl.kernel,
        out_type=jax.ShapeDtypeStruct((n_tokens, EMB), jnp.float32),
        mesh=mesh,
        scratch_types=[
            pltpu.VMEM((n_chunks, CHUNK), jnp.int32),
            pltpu.VMEM((GROUP_ROWS, EMB), jnp.float32),
            pltpu.VMEM((GROUP_ROWS, EMB), jnp.float32),
            pltpu.SemaphoreType.DMA,
            pltpu.SemaphoreType.DMA,
            pltpu.SemaphoreType.DMA,
        ],
        compiler_params=pltpu.CompilerParams(use_tc_tiling_on_sc=False),
    )
    def k(idx_hbm, table_hbm, out_hbm, idx_v, buf_a, buf_b, gsem, osem_a,
          osem_b):
        wid = lax.axis_index("s") * NC + lax.axis_index("c")
        base = wid * per_w
        pltpu.sync_copy(idx_hbm.at[wid], idx_v)

        def out_copy(buf, g, sem):
            return pltpu.make_async_copy(
                buf, out_hbm.at[pl.ds(base + g * GROUP_ROWS, GROUP_ROWS)],
                sem)

        def fire_gathers(buf, g):
            for b in range(S):
                pltpu.async_copy(table_hbm.at[idx_v.at[g * S + b]],
                                 buf.at[pl.ds(b * CHUNK, CHUNK)], gsem)

        def drain_gathers(buf):
            pltpu.make_async_copy(table_hbm.at[pl.ds(0, GROUP_ROWS)], buf,
                                  gsem).wait()

        def pair(i, _):
            g_a = 2 * i
            g_b = 2 * i + 1

            @pl.when(i > 0)
            def _():
                out_copy(buf_a, g_a - 2, osem_a).wait()

            fire_gathers(buf_a, g_a)

            @pl.when(i > 0)
            def _():
                out_copy(buf_b, g_b - 2, osem_b).wait()

            drain_gathers(buf_a)
            out_copy(buf_a, g_a, osem_a).start()
            fire_gathers(buf_b, g_b)
            drain_gathers(buf_b)
            out_copy(buf_b, g_b, osem_b).start()
            return 0

        lax.fori_loop(0, pairs, pair, 0)
        out_copy(buf_a, n_groups - 2, osem_a).wait()
        out_copy(buf_b, n_groups - 1, osem_b).wait()

    return k


def kernel(X, word_table, pos_table):
    B, T = X.shape
    n_tokens = B * T
    fused = _fuse_tables(word_table[:MAX_LEN], pos_table)
    idx = X.reshape(NW, n_tokens // (NW * CHUNK), CHUNK)
    out = _gather_kernel(n_tokens)(idx, fused)
    return out.reshape(B, T, EMB)


# speedup vs baseline: 1.0026x; 1.0026x over previous
import functools

import jax
import jax.numpy as jnp
from jax import lax
from jax.experimental import pallas as pl
from jax.experimental.pallas import tpu as pltpu
from jax.experimental.pallas import tpu_sc as plsc

MAX_LEN = 8192
EMB = 64

NC = 2
NS = 16
NW = NC * NS

R = 4


def _fuse_body(w_ref, p_ref, o_ref):
    o_ref[...] = w_ref[...] + p_ref[...]


def _fuse_tables(word_head, pos_table):
    return pl.pallas_call(
        _fuse_body,
        out_shape=jax.ShapeDtypeStruct((MAX_LEN, EMB), jnp.float32),
    )(word_head, pos_table)


def _gather_kernel(B, T):
    assert B % NW == 0
    rows_w = B // NW
    assert rows_w % (2 * R) == 0
    n_groups = rows_w // R
    pairs = n_groups // 2

    mesh = plsc.VectorSubcoreMesh(core_axis_name="c", subcore_axis_name="s")

    @functools.partial(
        pl.kernel,
        out_type=jax.ShapeDtypeStruct((B, T, EMB), jnp.float32),
        mesh=mesh,
        scratch_types=[
            pltpu.VMEM((rows_w, T), jnp.int32),
            pltpu.VMEM((R, T, EMB), jnp.float32),
            pltpu.VMEM((R, T, EMB), jnp.float32),
            pltpu.SemaphoreType.DMA,
            pltpu.SemaphoreType.DMA,
            pltpu.SemaphoreType.DMA,
        ],
        compiler_params=pltpu.CompilerParams(use_tc_tiling_on_sc=False),
    )
    def k(idx_hbm, table_hbm, out_hbm, idx_v, buf_a, buf_b, gsem, osem_a,
          osem_b):
        wid = lax.axis_index("s") * NC + lax.axis_index("c")
        row0 = wid * rows_w
        pltpu.sync_copy(idx_hbm.at[pl.ds(row0, rows_w)], idx_v)

        def out_copy(buf, g, sem):
            return pltpu.make_async_copy(
                buf, out_hbm.at[pl.ds(row0 + g * R, R)], sem)

        def fire_gathers(buf, g):
            for r in range(R):
                pltpu.async_copy(table_hbm.at[idx_v.at[g * R + r]],
                                 buf.at[r], gsem)

        def drain_gathers(buf, g):
            pltpu.make_async_copy(out_hbm.at[pl.ds(row0 + g * R, R)], buf,
                                  gsem).wait()

        def pair(i, _):
            g_a = 2 * i
            g_b = 2 * i + 1

            @pl.when(i > 0)
            def _():
                out_copy(buf_a, g_a - 2, osem_a).wait()

            fire_gathers(buf_a, g_a)

            @pl.when(i > 0)
            def _():
                out_copy(buf_b, g_b - 2, osem_b).wait()

            drain_gathers(buf_a, g_a)
            out_copy(buf_a, g_a, osem_a).start()
            fire_gathers(buf_b, g_b)
            drain_gathers(buf_b, g_b)
            out_copy(buf_b, g_b, osem_b).start()
            return 0

        lax.fori_loop(0, pairs, pair, 0)
        out_copy(buf_a, n_groups - 2, osem_a).wait()
        out_copy(buf_b, n_groups - 1, osem_b).wait()

    return k


def kernel(X, word_table, pos_table):
    B, T = X.shape
    fused = _fuse_tables(word_table[:MAX_LEN], pos_table)
    return _gather_kernel(B, T)(X, fused)
